# baseline (speedup 1.0000x reference)
import functools

import jax
import jax.numpy as jnp
from jax import lax
from jax.experimental import pallas as pl
from jax.experimental.pallas import tpu as pltpu
from jax.experimental.pallas import tpu_sc as plsc

EMBED_DIM = 64
BATCH = 16384
PACKED_ROWS = 500000

NC = 2
NS = 16
L = 16
NW = NC * NS
BPW = BATCH // NW
CHUNK = 256
NCHUNK = BPW // CHUNK
CGROUPS = CHUNK // L

_mesh = plsc.VectorSubcoreMesh(
    core_axis_name="c", subcore_axis_name="s", num_cores=NC, num_subcores=NS
)


@functools.partial(
    pl.kernel,
    out_type=jax.ShapeDtypeStruct((BATCH,), jnp.float32),
    mesh=_mesh,
    scratch_types=[
        pltpu.VMEM((BPW,), jnp.int32),
        pltpu.VMEM((BPW,), jnp.int32),
        pltpu.VMEM((BPW,), jnp.int32),
        pltpu.VMEM((BPW,), jnp.int32),
        pltpu.VMEM((CHUNK, 128), jnp.float32),
        pltpu.VMEM((CHUNK, 128), jnp.float32),
        pltpu.VMEM((BPW,), jnp.float32),
        pltpu.SemaphoreType.DMA,
    ],
    compiler_params=pltpu.CompilerParams(needs_layout_passes=False),
)
def _sc_dot(uid_hbm, iid_hbm, ut_hbm, it_hbm, out_hbm,
            uid_v, iid_v, upk_v, ipk_v, urows_v, irows_v, out_v, sem):
    wid = lax.axis_index("s") * NC + lax.axis_index("c")
    base = wid * BPW
    pltpu.sync_copy(uid_hbm.at[pl.ds(base, BPW)], uid_v)
    pltpu.sync_copy(iid_hbm.at[pl.ds(base, BPW)], iid_v)

    for g in range(BPW // L):
        sl = pl.ds(g * L, L)
        upk_v[sl] = lax.shift_right_logical(uid_v[sl], 1)
        ipk_v[sl] = lax.shift_right_logical(iid_v[sl], 1)

    lane = lax.broadcasted_iota(jnp.int32, (L,), 0)

    for chunk in range(NCHUNK):
        coff = chunk * CHUNK
        cps = []
        for j in range(CHUNK // 128):
            isl = pl.ds(coff + j * 128, 128)
            dsl = pl.ds(j * 128, 128)
            cps.append(pltpu.async_copy(
                ut_hbm.at[upk_v.at[isl]], urows_v.at[dsl], sem))
            cps.append(pltpu.async_copy(
                it_hbm.at[ipk_v.at[isl]], irows_v.at[dsl], sem))
        for cp in cps:
            cp.wait()

        def group(g, carry):
            gsl = pl.ds(coff + g * L, L)
            rows = g * L + lane
            uoff = lax.shift_left(jnp.bitwise_and(uid_v[gsl], 1), 6)
            ioff = lax.shift_left(jnp.bitwise_and(iid_v[gsl], 1), 6)
            acc = jnp.zeros((L,), jnp.float32)
            for c in range(EMBED_DIM):
                ug = plsc.load_gather(urows_v, [rows, uoff + c])
                ig = plsc.load_gather(irows_v, [rows, ioff + c])
                acc = acc + ug * ig
            out_v[gsl] = acc
            return carry

        lax.fori_loop(0, CGROUPS, group, 0)

    pltpu.sync_copy(out_v, out_hbm.at[pl.ds(base, BPW)])


def kernel(user_ids, item_ids, user_table, item_table):
    return _sc_dot(user_ids.astype(jnp.int32), item_ids.astype(jnp.int32),
                   user_table.reshape(PACKED_ROWS, 128),
                   item_table.reshape(PACKED_ROWS, 128))


# speedup vs baseline: 1.2052x; 1.2052x over previous
import functools

import jax
import jax.numpy as jnp
from jax import lax
from jax.experimental import pallas as pl
from jax.experimental.pallas import tpu as pltpu
from jax.experimental.pallas import tpu_sc as plsc

EMBED_DIM = 64
BATCH = 16384

NC = 2
NS = 16
L = 16
NW = NC * NS
BPW = BATCH // NW
CHUNK = 256
NCHUNK = BPW // CHUNK
CGROUPS = CHUNK // L

_mesh = plsc.VectorSubcoreMesh(
    core_axis_name="c", subcore_axis_name="s", num_cores=NC, num_subcores=NS
)


@functools.partial(
    pl.kernel,
    out_type=jax.ShapeDtypeStruct((BATCH,), jnp.float32),
    mesh=_mesh,
    scratch_types=[
        pltpu.VMEM((BPW,), jnp.int32),
        pltpu.VMEM((BPW,), jnp.int32),
        pltpu.VMEM((CHUNK, 128), jnp.float32),
        pltpu.VMEM((CHUNK, 128), jnp.float32),
        pltpu.VMEM((BPW,), jnp.float32),
        pltpu.SemaphoreType.DMA,
    ],
    compiler_params=pltpu.CompilerParams(needs_layout_passes=False),
)
def _sc_dot(uid_hbm, iid_hbm, tab_hbm, out_hbm,
            uid_v, iid_v, urows_v, irows_v, out_v, sem):
    wid = lax.axis_index("s") * NC + lax.axis_index("c")
    base = wid * BPW
    pltpu.sync_copy(uid_hbm.at[pl.ds(base, BPW)], uid_v)
    pltpu.sync_copy(iid_hbm.at[pl.ds(base, BPW)], iid_v)

    lane = lax.broadcasted_iota(jnp.int32, (L,), 0)

    for chunk in range(NCHUNK):
        coff = chunk * CHUNK
        cps = []
        for j in range(CHUNK // 128):
            isl = pl.ds(coff + j * 128, 128)
            dsl = pl.ds(j * 128, 128)
            cps.append(pltpu.async_copy(
                tab_hbm.at[uid_v.at[isl]], urows_v.at[dsl], sem))
            cps.append(pltpu.async_copy(
                tab_hbm.at[iid_v.at[isl]], irows_v.at[dsl], sem))
        for cp in cps:
            cp.wait()

        def group(g, carry):
            rows = g * L + lane
            acc = jnp.zeros((L,), jnp.float32)
            for c in range(EMBED_DIM):
                ug = plsc.load_gather(urows_v, [rows, jnp.full((L,), c, jnp.int32)])
                ig = plsc.load_gather(irows_v, [rows, jnp.full((L,), 64 + c, jnp.int32)])
                acc = acc + ug * ig
            out_v[pl.ds(coff + g * L, L)] = acc
            return carry

        lax.fori_loop(0, CGROUPS, group, 0)

    pltpu.sync_copy(out_v, out_hbm.at[pl.ds(base, BPW)])


def kernel(user_ids, item_ids, user_table, item_table):
    tab = jnp.concatenate([user_table, item_table], axis=1)
    return _sc_dot(user_ids.astype(jnp.int32), item_ids.astype(jnp.int32), tab)


# speedup vs baseline: 1.4691x; 1.2190x over previous
import functools

import jax
import jax.numpy as jnp
from jax import lax
from jax.experimental import pallas as pl
from jax.experimental.pallas import tpu as pltpu
from jax.experimental.pallas import tpu_sc as plsc

EMBED_DIM = 64
BATCH = 16384

NC = 2
NS = 16
L = 16
NW = NC * NS
BPW = BATCH // NW
CHUNK = 32
NCHUNK = BPW // CHUNK
CGROUPS = CHUNK // L

_mesh = plsc.VectorSubcoreMesh(
    core_axis_name="c", subcore_axis_name="s", num_cores=NC, num_subcores=NS
)


@functools.partial(
    pl.kernel,
    out_type=jax.ShapeDtypeStruct((BATCH,), jnp.float32),
    mesh=_mesh,
    scratch_types=[
        pltpu.VMEM((BPW,), jnp.int32),
        pltpu.VMEM((BPW,), jnp.int32),
        pltpu.VMEM((CHUNK, 8, EMBED_DIM), jnp.float32),
        pltpu.VMEM((CHUNK, 8, EMBED_DIM), jnp.float32),
        pltpu.VMEM((BPW,), jnp.float32),
        pltpu.SemaphoreType.DMA,
    ],
    compiler_params=pltpu.CompilerParams(needs_layout_passes=False),
)
def _sc_dot(uid_hbm, iid_hbm, ut_hbm, it_hbm, out_hbm,
            uid_v, iid_v, ubuf_v, ibuf_v, out_v, sem):
    wid = lax.axis_index("s") * NC + lax.axis_index("c")
    base = wid * BPW
    pltpu.sync_copy(uid_hbm.at[pl.ds(base, BPW)], uid_v)
    pltpu.sync_copy(iid_hbm.at[pl.ds(base, BPW)], iid_v)

    lane = lax.broadcasted_iota(jnp.int32, (L,), 0)

    def chunk_body(chunk, carry):
        coff = chunk * CHUNK
        cps = []
        for g in range(CGROUPS):
            ublks = lax.shift_right_logical(uid_v[pl.ds(coff + g * L, L)], 3) * 8
            iblks = lax.shift_right_logical(iid_v[pl.ds(coff + g * L, L)], 3) * 8
            for j in range(L):
                ub = pl.multiple_of(ublks[j], 8)
                ib = pl.multiple_of(iblks[j], 8)
                cps.append(pltpu.async_copy(
                    ut_hbm.at[pl.ds(ub, 8)], ubuf_v.at[g * L + j], sem))
                cps.append(pltpu.async_copy(
                    it_hbm.at[pl.ds(ib, 8)], ibuf_v.at[g * L + j], sem))
        for cp in cps:
            cp.wait()

        for g in range(CGROUPS):
            gsl = pl.ds(coff + g * L, L)
            rows = g * L + lane
            usub = jnp.bitwise_and(uid_v[gsl], 7)
            isub = jnp.bitwise_and(iid_v[gsl], 7)
            acc = jnp.zeros((L,), jnp.float32)
            for c in range(EMBED_DIM):
                col = jnp.full((L,), c, jnp.int32)
                ug = plsc.load_gather(ubuf_v, [rows, usub, col])
                ig = plsc.load_gather(ibuf_v, [rows, isub, col])
                acc = acc + ug * ig
            out_v[gsl] = acc
        return carry

    lax.fori_loop(0, NCHUNK, chunk_body, 0)

    pltpu.sync_copy(out_v, out_hbm.at[pl.ds(base, BPW)])


def kernel(user_ids, item_ids, user_table, item_table):
    return _sc_dot(user_ids.astype(jnp.int32), item_ids.astype(jnp.int32),
                   user_table, item_table)
